# baseline (speedup 1.0000x reference)
import functools

import jax
import jax.numpy as jnp
from jax import lax
from jax.experimental import pallas as pl
from jax.experimental.pallas import tpu as pltpu
from jax.experimental.pallas import tpu_sc as plsc

_NC = 2
_NS = 16
_NW = _NC * _NS
_LANES = 16


def _rsqrt16(v):
    y = plsc.bitcast(v, jnp.int32)
    y = jnp.int32(0x5F3759DF) - (y >> 1)
    g = plsc.bitcast(y, jnp.float32)
    for _ in range(3):
        g = g * (1.5 - 0.5 * v * g * g)
    return g


def _fused_embed_ln(ids, table, pos, gamma, beta, b, l, eps=1e-5):
    n = ids.shape[0]
    e = table.shape[1]
    seq_chunk = 16
    chunk = seq_chunk * l
    per_w = n // _NW
    n_chunks = per_w // chunk
    groups = chunk // _LANES
    mesh = plsc.VectorSubcoreMesh(core_axis_name="c", subcore_axis_name="s")

    @functools.partial(
        pl.kernel,
        out_type=jax.ShapeDtypeStruct((b, l, e), jnp.float32),
        mesh=mesh,
        scratch_types=[
            pltpu.VMEM((chunk,), jnp.int32),
            pltpu.VMEM((chunk, e), jnp.float32),
            pltpu.VMEM((l, e), jnp.float32),
            pltpu.VMEM((e, _LANES), jnp.float32),
            pltpu.VMEM((e, _LANES), jnp.float32),
            pltpu.VMEM((e, _LANES), jnp.float32),
            pltpu.VMEM((1, e), jnp.float32),
            pltpu.VMEM((1, e), jnp.float32),
            pltpu.VMEM((seq_chunk, l, e), jnp.float32),
            pltpu.SemaphoreType.DMA,
        ],
        compiler_params=pltpu.CompilerParams(use_tc_tiling_on_sc=False,
                                             needs_layout_passes=False),
    )
    def k(ids_hbm, tab_hbm, pos_hbm, g_hbm, b_hbm, out_hbm,
          idx_v, rows_v, posv, x_t, gbc, bbc, gl, bl, ybuf, sem):
        wid = lax.axis_index("s") * _NC + lax.axis_index("c")
        base = wid * per_w
        seq_base = wid * (per_w // l)
        iota = lax.iota(jnp.int32, _LANES)
        zeros16 = jnp.zeros((_LANES,), jnp.int32)

        pltpu.sync_copy(pos_hbm, posv)
        pltpu.sync_copy(g_hbm, gl)
        pltpu.sync_copy(b_hbm, bl)
        for ee in range(e):
            ce = jnp.full((_LANES,), ee, jnp.int32)
            gbc[ee] = plsc.load_gather(gl, [zeros16, ce])
            bbc[ee] = plsc.load_gather(bl, [zeros16, ce])

        def chunk_body(ci, carry):
            tok0 = base + ci * chunk
            pltpu.sync_copy(ids_hbm.at[pl.ds(tok0, chunk)], idx_v)
            pltpu.async_copy(tab_hbm.at[idx_v], rows_v, sem).wait()

            def group_body(g, carry2):
                ridx = g * _LANES + iota
                lrow = lax.rem(ridx, jnp.int32(l))
                srow = lax.div(ridx, jnp.int32(l))
                s = jnp.zeros((_LANES,), jnp.float32)
                ss = jnp.zeros((_LANES,), jnp.float32)
                for ee in range(e):
                    ce = jnp.full((_LANES,), ee, jnp.int32)
                    x = (plsc.load_gather(rows_v, [ridx, ce])
                         + plsc.load_gather(posv, [lrow, ce]))
                    s = s + x
                    ss = ss + x * x
                    plsc.store_scatter(x_t, [ce, iota], x)
                mean = s * (1.0 / e)
                var = ss * (1.0 / e) - mean * mean
                rstd = _rsqrt16(var + eps)
                for ee in range(e):
                    ce = jnp.full((_LANES,), ee, jnp.int32)
                    t = (x_t[ee] - mean) * rstd
                    y = t * gbc[ee] + bbc[ee]
                    plsc.store_scatter(ybuf, [srow, lrow, ce], y)
                return carry2

            lax.fori_loop(0, groups, group_body, 0)
            pltpu.sync_copy(ybuf, out_hbm.at[pl.ds(seq_base + ci * seq_chunk,
                                                   seq_chunk)])
            return carry

        lax.fori_loop(0, n_chunks, chunk_body, 0)

    return k(ids, table, pos, gamma.reshape(1, e), beta.reshape(1, e))


def kernel(input_ids, tok_table, pos_table, ln_gamma, ln_beta):
    b, l = input_ids.shape
    e = tok_table.shape[1]
    ids = input_ids.astype(jnp.int32).reshape(-1)
    return _fused_embed_ln(ids, tok_table, pos_table[:l], ln_gamma, ln_beta,
                           b, l)


# speedup vs baseline: 1.8890x; 1.8890x over previous
import functools

import jax
import jax.numpy as jnp
from jax import lax
from jax.experimental import pallas as pl
from jax.experimental.pallas import tpu as pltpu
from jax.experimental.pallas import tpu_sc as plsc

_NC = 2
_NS = 16
_NW = _NC * _NS


def _sc_pair_gather(ids2, x, chunk):
    n = ids2.shape[0]
    d = x.shape[1]
    per_w = n // _NW
    n_chunks = per_w // chunk
    mesh = plsc.VectorSubcoreMesh(core_axis_name="c", subcore_axis_name="s")

    @functools.partial(
        pl.kernel,
        out_type=jax.ShapeDtypeStruct((n, d), jnp.float32),
        mesh=mesh,
        scratch_types=[
            pltpu.VMEM((chunk,), jnp.int32),
            pltpu.VMEM((chunk, d), jnp.float32),
            pltpu.SemaphoreType.DMA,
        ],
    )
    def k(ids_hbm, x_hbm, out_hbm, idx_v, rows_v, sem):
        wid = lax.axis_index("s") * _NC + lax.axis_index("c")
        base = wid * per_w

        def body(i, carry):
            off = base + i * chunk
            pltpu.sync_copy(ids_hbm.at[pl.ds(off, chunk)], idx_v)
            pltpu.async_copy(x_hbm.at[idx_v], rows_v, sem).wait()
            pltpu.sync_copy(rows_v, out_hbm.at[pl.ds(off, chunk)])
            return carry

        lax.fori_loop(0, n_chunks, body, 0)

    return k(ids2, x)


def _tc_epilogue(rows, ids2d, pos, gamma, beta, eps=1e-5):
    b, l = ids2d.shape
    d2 = rows.shape[1]
    e = d2 // 2
    sb = 64

    def body(r_ref, id_ref, pos_ref, g_ref, b_ref, o_ref):
        r3 = r_ref[...].reshape(sb, l, d2)
        par = (id_ref[...] & 1)[:, :, None]
        x = jnp.where(par == 1, r3[:, :, e:], r3[:, :, :e]) + pos_ref[...]
        mean = jnp.mean(x, axis=-1, keepdims=True)
        xc = x - mean
        var = jnp.mean(xc * xc, axis=-1, keepdims=True)
        o_ref[...] = xc * (lax.rsqrt(var + eps) * g_ref[...]) + b_ref[...]

    return pl.pallas_call(
        body,
        grid=(b // sb,),
        in_specs=[
            pl.BlockSpec((sb * l, d2), lambda i: (i, 0)),
            pl.BlockSpec((sb, l), lambda i: (i, 0)),
            pl.BlockSpec((1, l, e), lambda i: (0, 0, 0)),
            pl.BlockSpec((1, 1, e), lambda i: (0, 0, 0)),
            pl.BlockSpec((1, 1, e), lambda i: (0, 0, 0)),
        ],
        out_specs=pl.BlockSpec((sb, l, e), lambda i: (i, 0, 0)),
        out_shape=jax.ShapeDtypeStruct((b, l, e), jnp.float32),
    )(rows, ids2d, pos.reshape(1, l, e), gamma.reshape(1, 1, e),
      beta.reshape(1, 1, e))


def kernel(input_ids, tok_table, pos_table, ln_gamma, ln_beta):
    b, l = input_ids.shape
    e = tok_table.shape[1]
    ids = input_ids.astype(jnp.int32)
    x = tok_table.reshape(tok_table.shape[0] // 2, 2 * e)
    rows = _sc_pair_gather(ids.reshape(-1) >> 1, x, chunk=640)
    return _tc_epilogue(rows, ids, pos_table[:l], ln_gamma, ln_beta)
